# baseline (speedup 1.0000x reference)
import functools

import jax
import jax.numpy as jnp
from jax import lax
from jax.experimental import pallas as pl
from jax.experimental.pallas import tpu as pltpu
from jax.experimental.pallas import tpu_sc as plsc

NTOK = 1000
D = 128
NR = 50
NRP = 64
NORM = 9.0
B = 16384

NC, NS = 2, 16
NW = NC * NS
RPW = B // NW
CH = 128
NCH = RPW // CH

BNW = 4

BLK = 2048
GSTEPS = B // BLK


def _sc_gather(W, b, noise, target):
    mesh = plsc.VectorSubcoreMesh(
        core_axis_name="c", subcore_axis_name="s", num_cores=NC, num_subcores=NS
    )

    @functools.partial(
        pl.kernel,
        out_type=(
            jax.ShapeDtypeStruct((B, D), jnp.float32),
            jax.ShapeDtypeStruct((B, BNW), jnp.float32),
        ),
        mesh=mesh,
        compiler_params=pltpu.CompilerParams(
            needs_layout_passes=False, use_tc_tiling_on_sc=False),
        scratch_types=[
            pltpu.VMEM((NCH, CH), jnp.int32),
            pltpu.VMEM((CH, D), jnp.float32),
            pltpu.VMEM((CH, D), jnp.float32),
            pltpu.VMEM((CH, D), jnp.float32),
            pltpu.VMEM((CH, D), jnp.float32),
            pltpu.VMEM((RPW, BNW), jnp.float32),
            pltpu.VMEM((NTOK,), jnp.float32),
            pltpu.VMEM((NTOK,), jnp.float32),
            pltpu.SemaphoreType.DMA,
            pltpu.SemaphoreType.DMA,
            pltpu.SemaphoreType.DMA,
            pltpu.SemaphoreType.DMA,
            pltpu.SemaphoreType.DMA,
        ],
    )
    def k(w_hbm, b_hbm, nz_hbm, t_hbm,
          wr_hbm, bnt_hbm,
          idx_v, rows_a, rows_b, rows_c, rows_d, bnt_v,
          btab_v, ntab_v, sg0, sg1, sg2, sg3, semo):
        wid = lax.axis_index("s") * NC + lax.axis_index("c")
        base = wid * RPW
        pltpu.sync_copy(b_hbm, btab_v)
        pltpu.sync_copy(nz_hbm, ntab_v)
        for g in range(NCH):
            pltpu.sync_copy(t_hbm.at[pl.ds(base + g * CH, CH)], idx_v.at[g])
        bufs = [rows_a, rows_b, rows_c, rows_d]
        sems = [sg0, sg1, sg2, sg3]
        gath = [
            pltpu.async_copy(w_hbm.at[idx_v.at[g]], bufs[g], sems[g])
            for g in range(NCH)
        ]
        lane = lax.iota(jnp.int32, 16)
        col0 = jnp.zeros((16,), jnp.int32)
        col1 = col0 + 1
        for j in range(RPW // 16):
            g, o = divmod(j, CH // 16)
            tv = idx_v[g, pl.ds(o * 16, 16)]
            bv = plsc.load_gather(btab_v, [tv])
            nv = plsc.load_gather(ntab_v, [tv])
            row = j * 16 + lane
            plsc.store_scatter(bnt_v, [row, col0], bv)
            plsc.store_scatter(bnt_v, [row, col1], nv)
        outc = []
        for g in range(NCH):
            gath[g].wait()
            outc.append(pltpu.async_copy(
                bufs[g], wr_hbm.at[pl.ds(base + g * CH, CH)], semo))
        pltpu.sync_copy(bnt_v, bnt_hbm.at[pl.ds(base, RPW)])
        for c in outc:
            c.wait()

    return k(W, b, noise, target)


def _noise_body(x_ref, w_ref, ns_ref, bnz_ref, nl_ref, wn_s, aux_s):
    i = pl.program_id(0)

    @pl.when(i == 0)
    def _():
        ns_col = ns_ref[:, 0:1]
        v_iota = lax.broadcasted_iota(jnp.int32, (NRP, NTOK), 1)
        onehot = (v_iota == ns_col).astype(jnp.float32)
        wn_s[...] = lax.dot_general(
            onehot, w_ref[...], (((1,), (0,)), ((), ())),
            precision=lax.Precision.HIGHEST,
            preferred_element_type=jnp.float32)
        bnz64 = lax.dot_general(
            onehot, bnz_ref[...], (((1,), (0,)), ((), ())),
            precision=lax.Precision.HIGHEST,
            preferred_element_type=jnp.float32)
        k_iota = lax.broadcasted_iota(jnp.int32, (2, 8), 1)
        r_iota = lax.broadcasted_iota(jnp.int32, (2, 8), 0)
        sel = (k_iota == r_iota).astype(jnp.float32)
        aux_s[...] = lax.dot_general(
            sel, bnz64, (((1,), (1,)), ((), ())),
            precision=lax.Precision.HIGHEST,
            preferred_element_type=jnp.float32)

    x = x_ref[...]
    bn_row = aux_s[0:1, :]
    nz_row = aux_s[1:2, :]
    nlog = lax.dot_general(
        x, wn_s[...], (((1,), (1,)), ((), ())),
        precision=lax.Precision.HIGHEST,
        preferred_element_type=jnp.float32) + bn_row - NORM
    npb = jnp.exp(nlog)
    kn = NR * nz_row
    mask = lax.broadcasted_iota(jnp.int32, (1, NRP), 1) < NR
    terms = jnp.where(mask, jnp.log(kn) - jnp.log(npb + kn), 0.0)
    nl_ref[...] = jnp.sum(terms, axis=1, keepdims=True)


def _tc_noise(x, W, ns2d, bnz):
    return pl.pallas_call(
        _noise_body,
        grid=(GSTEPS,),
        in_specs=[
            pl.BlockSpec((BLK, D), lambda i: (i, 0)),
            pl.BlockSpec((NTOK, D), lambda i: (0, 0)),
            pl.BlockSpec((NRP, 8), lambda i: (0, 0)),
            pl.BlockSpec((NTOK, 8), lambda i: (0, 0)),
        ],
        out_specs=pl.BlockSpec((BLK, 1), lambda i: (i, 0)),
        out_shape=jax.ShapeDtypeStruct((B, 1), jnp.float32),
        scratch_shapes=[
            pltpu.VMEM((NRP, D), jnp.float32),
            pltpu.VMEM((2, NRP), jnp.float32),
        ],
    )(x, W, ns2d, bnz)


def _data_body(x_ref, wr_ref, bnt_ref, nl_ref, out_ref):
    i = pl.program_id(0)
    x = x_ref[...]
    wr = wr_ref[...]
    bt = bnt_ref[:, 0:1]
    nt = bnt_ref[:, 1:2]
    dlog = jnp.sum(x * wr, axis=1, keepdims=True) + bt - NORM
    dp = jnp.exp(dlog)
    rnn = dlog - jnp.log(dp + NR * nt)
    tot = jnp.sum(rnn + nl_ref[...], axis=0, keepdims=True)
    prev = jnp.where(i == 0, 0.0, out_ref[...])
    out_ref[...] = prev + tot

    @pl.when(i == GSTEPS - 1)
    def _():
        out_ref[...] = out_ref[...] * (-1.0 / B)


def _tc_data(x, wr, bnt, nl):
    return pl.pallas_call(
        _data_body,
        grid=(GSTEPS,),
        in_specs=[
            pl.BlockSpec((BLK, D), lambda i: (i, 0)),
            pl.BlockSpec((BLK, D), lambda i: (i, 0)),
            pl.BlockSpec((BLK, BNW), lambda i: (i, 0)),
            pl.BlockSpec((BLK, 1), lambda i: (i, 0)),
        ],
        out_specs=pl.BlockSpec((1, 1), lambda i: (0, 0)),
        out_shape=jax.ShapeDtypeStruct((1, 1), jnp.float32),
    )(x, wr, bnt, nl)


def kernel(input, target, W, b, noise, noise_samples):
    target = target.astype(jnp.int32)
    ns_pad = jnp.concatenate(
        [noise_samples.astype(jnp.int32), jnp.zeros((NRP - NR,), jnp.int32)])
    ns2d = jnp.broadcast_to(ns_pad[:, None], (NRP, 8))
    bnz = jnp.concatenate(
        [b[:, None], noise[:, None], jnp.zeros((NTOK, 6), jnp.float32)],
        axis=1)
    wr, bnt = _sc_gather(W, b, noise, target)
    nl = _tc_noise(input, W, ns2d, bnz)
    out = _tc_data(input, wr, bnt, nl)
    return out[0, 0]


# speedup vs baseline: 1.0362x; 1.0362x over previous
import functools

import jax
import jax.numpy as jnp
from jax import lax
from jax.experimental import pallas as pl
from jax.experimental.pallas import tpu as pltpu
from jax.experimental.pallas import tpu_sc as plsc

NTOK = 1000
D = 128
NR = 50
NRP = 64
NORM = 9.0
B = 16384

NC, NS = 2, 16
NW = NC * NS
RPW = B // NW
CH = 128
NCH = RPW // CH

BNW = 4

BLK = 2048
GSTEPS = B // BLK


def _sc_gather(W, b, noise, target):
    mesh = plsc.VectorSubcoreMesh(
        core_axis_name="c", subcore_axis_name="s", num_cores=NC, num_subcores=NS
    )

    @functools.partial(
        pl.kernel,
        out_type=(
            jax.ShapeDtypeStruct((B, D), jnp.float32),
            jax.ShapeDtypeStruct((B, BNW), jnp.float32),
        ),
        mesh=mesh,
        compiler_params=pltpu.CompilerParams(needs_layout_passes=False),
        scratch_types=[
            pltpu.VMEM((NCH, CH), jnp.int32),
            pltpu.VMEM((CH, D), jnp.float32),
            pltpu.VMEM((CH, D), jnp.float32),
            pltpu.VMEM((RPW, BNW), jnp.float32),
            pltpu.VMEM((NTOK,), jnp.float32),
            pltpu.VMEM((NTOK,), jnp.float32),
            pltpu.SemaphoreType.DMA,
            pltpu.SemaphoreType.DMA,
        ],
    )
    def k(w_hbm, b_hbm, nz_hbm, t_hbm,
          wr_hbm, bnt_hbm,
          idx_v, rows_a, rows_b, bnt_v,
          btab_v, ntab_v, sg0, sg1):
        wid = lax.axis_index("s") * NC + lax.axis_index("c")
        base = wid * RPW
        pltpu.sync_copy(b_hbm, btab_v)
        pltpu.sync_copy(nz_hbm, ntab_v)
        for g in range(NCH):
            pltpu.sync_copy(t_hbm.at[pl.ds(base + g * CH, CH)], idx_v.at[g])
        bufs = [rows_a, rows_b]
        sems = [sg0, sg1]
        lane = lax.iota(jnp.int32, 16)
        col0 = jnp.zeros((16,), jnp.int32)
        col1 = col0 + 1
        gath = [pltpu.async_copy(w_hbm.at[idx_v.at[0]], bufs[0], sems[0]),
                None]
        for j in range(RPW // 16):
            g, o = divmod(j, CH // 16)
            tv = idx_v[g, pl.ds(o * 16, 16)]
            bv = plsc.load_gather(btab_v, [tv])
            nv = plsc.load_gather(ntab_v, [tv])
            row = j * 16 + lane
            plsc.store_scatter(bnt_v, [row, col0], bv)
            plsc.store_scatter(bnt_v, [row, col1], nv)
        for g in range(NCH):
            i = g % 2
            gath[i].wait()
            if g + 1 < NCH:
                gath[(g + 1) % 2] = pltpu.async_copy(
                    w_hbm.at[idx_v.at[g + 1]], bufs[(g + 1) % 2],
                    sems[(g + 1) % 2])
            pltpu.sync_copy(bufs[i], wr_hbm.at[pl.ds(base + g * CH, CH)])
        pltpu.sync_copy(bnt_v, bnt_hbm.at[pl.ds(base, RPW)])

    return k(W, b, noise, target)


def _noise_body(x_ref, w_ref, ns_ref, bnz_ref, nl_ref, wn_s, aux_s):
    i = pl.program_id(0)

    @pl.when(i == 0)
    def _():
        ns_col = ns_ref[:, 0:1]
        v_iota = lax.broadcasted_iota(jnp.int32, (NRP, NTOK), 1)
        onehot = (v_iota == ns_col).astype(jnp.float32)
        wn_s[...] = lax.dot_general(
            onehot, w_ref[...], (((1,), (0,)), ((), ())),
            precision=lax.Precision.HIGHEST,
            preferred_element_type=jnp.float32)
        bnz64 = lax.dot_general(
            onehot, bnz_ref[...], (((1,), (0,)), ((), ())),
            precision=lax.Precision.HIGHEST,
            preferred_element_type=jnp.float32)
        k_iota = lax.broadcasted_iota(jnp.int32, (2, 8), 1)
        r_iota = lax.broadcasted_iota(jnp.int32, (2, 8), 0)
        sel = (k_iota == r_iota).astype(jnp.float32)
        aux_s[...] = lax.dot_general(
            sel, bnz64, (((1,), (1,)), ((), ())),
            precision=lax.Precision.HIGHEST,
            preferred_element_type=jnp.float32)

    x = x_ref[...]
    bn_row = aux_s[0:1, :]
    nz_row = aux_s[1:2, :]
    nlog = lax.dot_general(
        x, wn_s[...], (((1,), (1,)), ((), ())),
        precision=lax.Precision.DEFAULT,
        preferred_element_type=jnp.float32) + bn_row - NORM
    npb = jnp.exp(nlog)
    kn = NR * nz_row
    mask = lax.broadcasted_iota(jnp.int32, (1, NRP), 1) < NR
    terms = jnp.where(mask, jnp.log(kn) - jnp.log(npb + kn), 0.0)
    nl_ref[...] = jnp.sum(terms, axis=1, keepdims=True)


def _tc_noise(x, W, ns2d, bnz):
    return pl.pallas_call(
        _noise_body,
        grid=(GSTEPS,),
        in_specs=[
            pl.BlockSpec((BLK, D), lambda i: (i, 0)),
            pl.BlockSpec((NTOK, D), lambda i: (0, 0)),
            pl.BlockSpec((NRP, 8), lambda i: (0, 0)),
            pl.BlockSpec((NTOK, 8), lambda i: (0, 0)),
        ],
        out_specs=pl.BlockSpec((BLK, 1), lambda i: (i, 0)),
        out_shape=jax.ShapeDtypeStruct((B, 1), jnp.float32),
        scratch_shapes=[
            pltpu.VMEM((NRP, D), jnp.float32),
            pltpu.VMEM((2, NRP), jnp.float32),
        ],
    )(x, W, ns2d, bnz)


def _data_body(x_ref, wr_ref, bnt_ref, nl_ref, out_ref):
    i = pl.program_id(0)
    x = x_ref[...]
    wr = wr_ref[...]
    bt = bnt_ref[:, 0:1]
    nt = bnt_ref[:, 1:2]
    ones = jnp.ones((D, 1), jnp.float32)
    dlog = lax.dot_general(
        x * wr, ones, (((1,), (0,)), ((), ())),
        precision=lax.Precision.HIGHEST,
        preferred_element_type=jnp.float32) + bt - NORM
    dp = jnp.exp(dlog)
    rnn = dlog - jnp.log(dp + NR * nt)
    tot = jnp.sum(rnn + nl_ref[...], axis=0, keepdims=True)
    prev = jnp.where(i == 0, 0.0, out_ref[...])
    out_ref[...] = prev + tot

    @pl.when(i == GSTEPS - 1)
    def _():
        out_ref[...] = out_ref[...] * (-1.0 / B)


def _tc_data(x, wr, bnt, nl):
    return pl.pallas_call(
        _data_body,
        grid=(GSTEPS,),
        in_specs=[
            pl.BlockSpec((BLK, D), lambda i: (i, 0)),
            pl.BlockSpec((BLK, D), lambda i: (i, 0)),
            pl.BlockSpec((BLK, BNW), lambda i: (i, 0)),
            pl.BlockSpec((BLK, 1), lambda i: (i, 0)),
        ],
        out_specs=pl.BlockSpec((1, 1), lambda i: (0, 0)),
        out_shape=jax.ShapeDtypeStruct((1, 1), jnp.float32),
    )(x, wr, bnt, nl)


def kernel(input, target, W, b, noise, noise_samples):
    target = target.astype(jnp.int32)
    ns_pad = jnp.concatenate(
        [noise_samples.astype(jnp.int32), jnp.zeros((NRP - NR,), jnp.int32)])
    ns2d = jnp.broadcast_to(ns_pad[:, None], (NRP, 8))
    bnz = jnp.concatenate(
        [b[:, None], noise[:, None], jnp.zeros((NTOK, 6), jnp.float32)],
        axis=1)
    wr, bnt = _sc_gather(W, b, noise, target)
    nl = _tc_noise(input, W, ns2d, bnz)
    out = _tc_data(input, wr, bnt, nl)
    return out[0, 0]


# speedup vs baseline: 1.3412x; 1.2943x over previous
import functools

import jax
import jax.numpy as jnp
from jax import lax
from jax.experimental import pallas as pl
from jax.experimental.pallas import tpu as pltpu
from jax.experimental.pallas import tpu_sc as plsc

NTOK = 1000
D = 128
NR = 50
NRP = 64
NORM = 9.0
B = 16384

NC, NS = 2, 16
NW = NC * NS
RPW = B // NW
CH = 128
NCH = RPW // CH

BNW = 8

BLK = 4096
GSTEPS = B // BLK


def _sc_gather(W, b, noise, target):
    mesh = plsc.VectorSubcoreMesh(
        core_axis_name="c", subcore_axis_name="s", num_cores=NC, num_subcores=NS
    )

    @functools.partial(
        pl.kernel,
        out_type=(
            jax.ShapeDtypeStruct((B, D), jnp.float32),
            jax.ShapeDtypeStruct((BNW, B), jnp.float32),
        ),
        mesh=mesh,
        compiler_params=pltpu.CompilerParams(needs_layout_passes=False),
        scratch_types=[
            pltpu.VMEM((NCH, CH), jnp.int32),
            pltpu.VMEM((CH, D), jnp.float32),
            pltpu.VMEM((CH, D), jnp.float32),
            pltpu.VMEM((BNW, RPW), jnp.float32),
            pltpu.VMEM((NTOK,), jnp.float32),
            pltpu.VMEM((NTOK,), jnp.float32),
            pltpu.SemaphoreType.DMA,
            pltpu.SemaphoreType.DMA,
            pltpu.SemaphoreType.DMA,
            pltpu.SemaphoreType.DMA,
        ],
    )
    def k(w_hbm, b_hbm, nz_hbm, t_hbm,
          wr_hbm, bnt_hbm,
          idx_v, rows_a, rows_b, bnt_v,
          btab_v, ntab_v, semi, semt, sg0, sg1):
        wid = lax.axis_index("s") * NC + lax.axis_index("c")
        base = wid * RPW
        idxc = [
            pltpu.async_copy(
                t_hbm.at[pl.ds(base + g * CH, CH)], idx_v.at[g], semi)
            for g in range(NCH)
        ]
        tbc = [pltpu.async_copy(b_hbm, btab_v, semt),
               pltpu.async_copy(nz_hbm, ntab_v, semt)]
        for c in idxc:
            c.wait()
        bufs = [rows_a, rows_b]
        sems = [sg0, sg1]
        gath = [pltpu.async_copy(w_hbm.at[idx_v.at[0]], bufs[0], sems[0]),
                None]
        for c in tbc:
            c.wait()
        lane = lax.iota(jnp.int32, 16)
        col0 = jnp.zeros((16,), jnp.int32)
        col1 = col0 + 1
        for j in range(RPW // 16):
            g, o = divmod(j, CH // 16)
            tv = idx_v[g, pl.ds(o * 16, 16)]
            bv = plsc.load_gather(btab_v, [tv])
            nv = plsc.load_gather(ntab_v, [tv])
            row = j * 16 + lane
            plsc.store_scatter(bnt_v, [col0, row], bv)
            plsc.store_scatter(bnt_v, [col1, row], nv)
        for g in range(NCH):
            i = g % 2
            gath[i].wait()
            if g + 1 < NCH:
                gath[(g + 1) % 2] = pltpu.async_copy(
                    w_hbm.at[idx_v.at[g + 1]], bufs[(g + 1) % 2],
                    sems[(g + 1) % 2])
            pltpu.sync_copy(bufs[i], wr_hbm.at[pl.ds(base + g * CH, CH)])
        pltpu.sync_copy(bnt_v, bnt_hbm.at[:, pl.ds(base, RPW)])

    return k(W, b, noise, target)


def _noise_body(x_ref, w_ref, ns_ref, bnz_ref, out_ref, wn_s, aux_s):
    i = pl.program_id(0)

    @pl.when(i == 0)
    def _():
        ns_col = ns_ref[:, 0:1]
        v_iota = lax.broadcasted_iota(jnp.int32, (NRP, NTOK), 1)
        onehot = (v_iota == ns_col).astype(jnp.float32)
        wn_s[...] = lax.dot_general(
            onehot, w_ref[...], (((1,), (0,)), ((), ())),
            precision=lax.Precision.HIGHEST,
            preferred_element_type=jnp.float32)
        bnz64 = lax.dot_general(
            onehot, bnz_ref[...], (((1,), (0,)), ((), ())),
            precision=lax.Precision.HIGHEST,
            preferred_element_type=jnp.float32)
        k_iota = lax.broadcasted_iota(jnp.int32, (2, 8), 1)
        r_iota = lax.broadcasted_iota(jnp.int32, (2, 8), 0)
        sel = (k_iota == r_iota).astype(jnp.float32)
        aux_s[...] = lax.dot_general(
            sel, bnz64, (((1,), (1,)), ((), ())),
            precision=lax.Precision.HIGHEST,
            preferred_element_type=jnp.float32)

    x = x_ref[...]
    bn_row = aux_s[0:1, :]
    nz_row = aux_s[1:2, :]
    nlog = lax.dot_general(
        x, wn_s[...], (((1,), (1,)), ((), ())),
        precision=lax.Precision.DEFAULT,
        preferred_element_type=jnp.float32) + bn_row - NORM
    npb = jnp.exp(nlog)
    kn = NR * nz_row
    mask = lax.broadcasted_iota(jnp.int32, (1, NRP), 1) < NR
    terms = jnp.where(mask, jnp.log(kn) - jnp.log(npb + kn), 0.0)
    tot = jnp.sum(jnp.sum(terms, axis=1, keepdims=True),
                  axis=0, keepdims=True)
    prev = jnp.where(i == 0, 0.0, out_ref[...])
    out_ref[...] = prev + tot


def _tc_noise(x, W, ns2d, bnz):
    return pl.pallas_call(
        _noise_body,
        grid=(GSTEPS,),
        in_specs=[
            pl.BlockSpec((BLK, D), lambda i: (i, 0)),
            pl.BlockSpec((NTOK, D), lambda i: (0, 0)),
            pl.BlockSpec((NRP, 8), lambda i: (0, 0)),
            pl.BlockSpec((NTOK, 8), lambda i: (0, 0)),
        ],
        out_specs=pl.BlockSpec((1, 1), lambda i: (0, 0)),
        out_shape=jax.ShapeDtypeStruct((1, 1), jnp.float32),
        scratch_shapes=[
            pltpu.VMEM((NRP, D), jnp.float32),
            pltpu.VMEM((2, NRP), jnp.float32),
        ],
    )(x, W, ns2d, bnz)


def _data_body(x_ref, wr_ref, bnt_ref, sn_ref, out_ref):
    i = pl.program_id(0)
    xwr = x_ref[...] * wr_ref[...]
    ones = jnp.ones((1, D), jnp.float32)
    bt = bnt_ref[0:1, :]
    nt = bnt_ref[1:2, :]
    dlog = lax.dot_general(
        ones, xwr, (((1,), (1,)), ((), ())),
        precision=lax.Precision.HIGHEST,
        preferred_element_type=jnp.float32) + bt - NORM
    dp = jnp.exp(dlog)
    rnn = dlog - jnp.log(dp + NR * nt)
    tot = jnp.sum(rnn, axis=1, keepdims=True)
    prev = jnp.where(i == 0, 0.0, out_ref[...])
    out_ref[...] = prev + tot

    @pl.when(i == GSTEPS - 1)
    def _():
        out_ref[...] = (out_ref[...] + sn_ref[...]) * (-1.0 / B)


def _tc_data(x, wr, bnt, sn):
    return pl.pallas_call(
        _data_body,
        grid=(GSTEPS,),
        in_specs=[
            pl.BlockSpec((BLK, D), lambda i: (i, 0)),
            pl.BlockSpec((BLK, D), lambda i: (i, 0)),
            pl.BlockSpec((BNW, BLK), lambda i: (0, i)),
            pl.BlockSpec((1, 1), lambda i: (0, 0)),
        ],
        out_specs=pl.BlockSpec((1, 1), lambda i: (0, 0)),
        out_shape=jax.ShapeDtypeStruct((1, 1), jnp.float32),
    )(x, wr, bnt, sn)


def kernel(input, target, W, b, noise, noise_samples):
    target = target.astype(jnp.int32)
    ns_pad = jnp.concatenate(
        [noise_samples.astype(jnp.int32), jnp.zeros((NRP - NR,), jnp.int32)])
    ns2d = jnp.broadcast_to(ns_pad[:, None], (NRP, 8))
    bnz = jnp.concatenate(
        [b[:, None], noise[:, None], jnp.zeros((NTOK, 6), jnp.float32)],
        axis=1)
    wr, bnt = _sc_gather(W, b, noise, target)
    sn = _tc_noise(input, W, ns2d, bnz)
    out = _tc_data(input, wr, bnt, sn)
    return out[0, 0]
